# baseline (speedup 1.0000x reference)
import jax
import jax.numpy as jnp
from jax import lax
from jax.experimental import pallas as pl
from jax.experimental.pallas import tpu as pltpu
from jax.experimental.pallas import tpu_sc as plsc

N_NODES = 10000
D = 128
E = 320000
N_PAD = 10240
EPW = E // 16
CHUNK = 128
CHUNKS_C0 = 89
CHUNKS_C1 = 67
C1_OFF = CHUNKS_C0 * CHUNK
TAIL_OFF = C1_OFF + CHUNKS_C1 * CHUNK
TAIL = EPW - TAIL_OFF
DUMMY_ROW = N_PAD - 1
ROWS_PER_SUB = N_PAD // 16


def _make_sc_agg(with_counts):
    mesh = plsc.VectorSubcoreMesh(core_axis_name="c", subcore_axis_name="s")
    out_type = [jax.ShapeDtypeStruct((2, N_PAD, D), jnp.float32)]
    scratch = [
        pltpu.VMEM((8, CHUNK), jnp.int32),
        pltpu.VMEM((8, CHUNK), jnp.int32),
        pltpu.SemaphoreType.DMA,
        pltpu.VMEM((2, CHUNK), jnp.int32),
        pltpu.VMEM((CHUNK, D), jnp.float32),
        pltpu.VMEM((CHUNK, D), jnp.float32),
        pltpu.VMEM_SHARED((N_PAD, D), jnp.float32),
        pltpu.SemaphoreType.DMA,
        pltpu.SemaphoreType.DMA,
    ]
    if with_counts:
        out_type.append(jax.ShapeDtypeStruct((2, 1, N_PAD), jnp.float32))
        scratch += [
            pltpu.VMEM((CHUNK,), jnp.float32),
            pltpu.VMEM_SHARED((N_PAD,), jnp.float32),
        ]

    def body(x_hbm, src_hbm, dst_hbm, tails_hbm, z2_hbm, z1_hbm, *rest):
        if with_counts:
            (out_hbm, cnt_hbm, src_r, dst_r, sem_i, tail_v, rows_a, rows_b,
             acc_sh, sem_a, sem_b, ones_v, cnt_sh) = rest
        else:
            (out_hbm, src_r, dst_r, sem_i, tail_v, rows_a, rows_b,
             acc_sh, sem_a, sem_b) = rest
        c = lax.axis_index("c")
        s = lax.axis_index("s")
        row0 = s * ROWS_PER_SUB
        rows_sl = pl.ds(row0, ROWS_PER_SUB)
        base = s * EPW + c * C1_OFF
        chunks_mine = jnp.where(c == 0, CHUNKS_C0, CHUNKS_C1)
        pltpu.sync_copy(z2_hbm.at[rows_sl], acc_sh.at[rows_sl])
        if with_counts:
            pltpu.sync_copy(z1_hbm.at[rows_sl], cnt_sh.at[rows_sl])
            for i in range(CHUNK // 16):
                ones_v[pl.ds(i * 16, 16)] = jnp.ones((16,), jnp.float32)

        def idx_load(j, slot):
            sl = pl.ds(base + j * CHUNK, CHUNK)
            pltpu.make_async_copy(src_hbm.at[sl], src_r.at[slot],
                                  sem_i).start()
            pltpu.make_async_copy(dst_hbm.at[sl], dst_r.at[slot],
                                  sem_i).start()

        for t in range(8):
            idx_load(t, t)

        @pl.when(c == 1)
        def _():
            pltpu.sync_copy(tails_hbm.at[s], tail_v)
        plsc.subcore_barrier()

        def idx_wait(slot):
            pltpu.make_async_copy(src_hbm.at[pl.ds(0, CHUNK)],
                                  src_r.at[slot], sem_i).wait()
            pltpu.make_async_copy(dst_hbm.at[pl.ds(0, CHUNK)],
                                  dst_r.at[slot], sem_i).wait()

        def gather_start(sbuf, buf, sem):
            pltpu.make_async_copy(x_hbm.at[sbuf], buf, sem).start()

        def gather_wait(sbuf, buf, sem):
            pltpu.make_async_copy(x_hbm.at[sbuf], buf, sem).wait()

        def scatter(j, dbuf, buf):
            pltpu.sync_copy(buf, acc_sh.at[dbuf], add=True)
            if with_counts:
                pltpu.sync_copy(ones_v, cnt_sh.at[dbuf], add=True)

            @pl.when(j + 8 < chunks_mine)
            def _():
                idx_load(j + 8, jnp.bitwise_and(j, 7))

        @pl.when(c == 1)
        def _():
            gather_start(tail_v.at[0], rows_a, sem_a)
            gather_wait(tail_v.at[0], rows_a, sem_a)
            pltpu.sync_copy(rows_a, acc_sh.at[tail_v.at[1]], add=True)
            if with_counts:
                pltpu.sync_copy(ones_v, cnt_sh.at[tail_v.at[1]], add=True)

        n_pairs = jnp.where(c == 0, (CHUNKS_C0 - 1) // 2,
                            (CHUNKS_C1 - 1) // 2)
        idx_wait(0)
        gather_start(src_r.at[0], rows_a, sem_a)

        def step(i, carry):
            j = 2 * i
            sl_a = jnp.bitwise_and(j, 7)
            sl_b = jnp.bitwise_and(j + 1, 7)
            sl_a2 = jnp.bitwise_and(j + 2, 7)
            idx_wait(sl_b)
            gather_start(src_r.at[sl_b], rows_b, sem_b)
            gather_wait(src_r.at[sl_a], rows_a, sem_a)
            scatter(j, dst_r.at[sl_a], rows_a)
            idx_wait(sl_a2)
            gather_start(src_r.at[sl_a2], rows_a, sem_a)
            gather_wait(src_r.at[sl_b], rows_b, sem_b)
            scatter(j + 1, dst_r.at[sl_b], rows_b)
            return carry

        lax.fori_loop(0, n_pairs, step, 0)
        last_sl = jnp.bitwise_and(chunks_mine - 1, 7)
        gather_wait(src_r.at[last_sl], rows_a, sem_a)
        scatter(chunks_mine - 1, dst_r.at[last_sl], rows_a)

        plsc.subcore_barrier()
        pltpu.sync_copy(acc_sh.at[rows_sl], out_hbm.at[c, rows_sl])
        if with_counts:
            pltpu.sync_copy(cnt_sh.at[rows_sl], cnt_hbm.at[c, 0, rows_sl])

    return pl.kernel(body, mesh=mesh, out_type=out_type,
                     scratch_types=scratch)


_sc_agg_counts = _make_sc_agg(True)
_sc_agg = _make_sc_agg(False)


def _make_tc_layer(relu):
    def body(p_ref, cnt_ref, x_ref, wl_ref, b_ref, wr_ref, o_ref):
        ssum = p_ref[0] + p_ref[1]
        csum = cnt_ref[0, 0] + cnt_ref[1, 0]
        inv = 1.0 / jnp.maximum(csum, 1.0)
        mean = (ssum * lax.broadcast_in_dim(inv, (N_PAD, D), (0,)))
        mean = mean[:N_NODES]
        acc = lax.dot_general(mean, wl_ref[...], (((1,), (1,)), ((), ())),
                              preferred_element_type=jnp.float32)
        acc = acc + lax.dot_general(x_ref[...], wr_ref[...],
                                    (((1,), (1,)), ((), ())),
                                    preferred_element_type=jnp.float32)
        acc = acc + b_ref[...]
        if relu:
            acc = jnp.maximum(acc, 0.0)
        o_ref[...] = acc

    return pl.pallas_call(
        body,
        out_shape=jax.ShapeDtypeStruct((N_NODES, D), jnp.float32),
    )


_tc_layer_relu = _make_tc_layer(True)
_tc_layer_lin = _make_tc_layer(False)


def kernel(x, edge_index, W1l, b1l, W1r, W2l, b2l, W2r):
    src = edge_index[0].astype(jnp.int32)
    dst = edge_index[1].astype(jnp.int32)
    tail_src = jnp.concatenate(
        [src.reshape(16, EPW)[:, TAIL_OFF:],
         jnp.zeros((16, CHUNK - TAIL), jnp.int32)], axis=1)
    tail_dst = jnp.concatenate(
        [dst.reshape(16, EPW)[:, TAIL_OFF:],
         jnp.full((16, CHUNK - TAIL), DUMMY_ROW, jnp.int32)], axis=1)
    tails = jnp.stack([tail_src, tail_dst], axis=1)

    z2 = jnp.zeros((N_PAD, D), jnp.float32)
    z1 = jnp.zeros((N_PAD,), jnp.float32)
    b1 = b1l.reshape(1, D)
    b2 = b2l.reshape(1, D)

    p1, cnt = _sc_agg_counts(x, src, dst, tails, z2, z1)
    h = _tc_layer_relu(p1, cnt, x, W1l, b1, W1r)
    (p2,) = _sc_agg(h, src, dst, tails, z2, z1)
    return _tc_layer_lin(p2, cnt, h, W2l, b2, W2r)


# speedup vs baseline: 1.0146x; 1.0146x over previous
import jax
import jax.numpy as jnp
from jax import lax
from jax.experimental import pallas as pl
from jax.experimental.pallas import tpu as pltpu
from jax.experimental.pallas import tpu_sc as plsc

N_NODES = 10000
D = 128
E = 320000
N_PAD = 10240
EPW = E // 16
CHUNK = 128
CHUNKS_C0 = 93
CHUNKS_C1 = 63
C1_OFF = CHUNKS_C0 * CHUNK
TAIL_OFF = C1_OFF + CHUNKS_C1 * CHUNK
TAIL = EPW - TAIL_OFF
DUMMY_ROW = N_PAD - 1
ROWS_PER_SUB = N_PAD // 16


def _make_sc_agg(with_counts):
    mesh = plsc.VectorSubcoreMesh(core_axis_name="c", subcore_axis_name="s")
    out_type = [jax.ShapeDtypeStruct((2, N_PAD, D), jnp.float32)]
    scratch = [
        pltpu.VMEM((8, CHUNK), jnp.int32),
        pltpu.VMEM((8, CHUNK), jnp.int32),
        pltpu.SemaphoreType.DMA,
        pltpu.VMEM((2, CHUNK), jnp.int32),
        pltpu.VMEM((CHUNK, D), jnp.float32),
        pltpu.VMEM((CHUNK, D), jnp.float32),
        pltpu.VMEM_SHARED((N_PAD, D), jnp.float32),
        pltpu.SemaphoreType.DMA,
        pltpu.SemaphoreType.DMA,
    ]
    if with_counts:
        out_type.append(jax.ShapeDtypeStruct((2, 1, N_PAD), jnp.float32))
        scratch += [
            pltpu.VMEM((CHUNK,), jnp.float32),
            pltpu.VMEM_SHARED((N_PAD,), jnp.float32),
        ]

    def body(x_hbm, src_hbm, dst_hbm, tails_hbm, z2_hbm, z1_hbm, *rest):
        if with_counts:
            (out_hbm, cnt_hbm, src_r, dst_r, sem_i, tail_v, rows_a, rows_b,
             acc_sh, sem_a, sem_b, ones_v, cnt_sh) = rest
        else:
            (out_hbm, src_r, dst_r, sem_i, tail_v, rows_a, rows_b,
             acc_sh, sem_a, sem_b) = rest
        c = lax.axis_index("c")
        s = lax.axis_index("s")
        row0 = s * ROWS_PER_SUB
        rows_sl = pl.ds(row0, ROWS_PER_SUB)
        base = s * EPW + c * C1_OFF
        chunks_mine = jnp.where(c == 0, CHUNKS_C0, CHUNKS_C1)
        pltpu.sync_copy(z2_hbm.at[rows_sl], acc_sh.at[rows_sl])
        if with_counts:
            pltpu.sync_copy(z1_hbm.at[rows_sl], cnt_sh.at[rows_sl])
            for i in range(CHUNK // 16):
                ones_v[pl.ds(i * 16, 16)] = jnp.ones((16,), jnp.float32)

        def idx_load(j, slot):
            sl = pl.ds(base + j * CHUNK, CHUNK)
            pltpu.make_async_copy(src_hbm.at[sl], src_r.at[slot],
                                  sem_i).start()
            pltpu.make_async_copy(dst_hbm.at[sl], dst_r.at[slot],
                                  sem_i).start()

        for t in range(8):
            idx_load(t, t)

        @pl.when(c == 1)
        def _():
            pltpu.sync_copy(tails_hbm.at[s], tail_v)
        plsc.subcore_barrier()

        def idx_wait(slot):
            pltpu.make_async_copy(src_hbm.at[pl.ds(0, CHUNK)],
                                  src_r.at[slot], sem_i).wait()
            pltpu.make_async_copy(dst_hbm.at[pl.ds(0, CHUNK)],
                                  dst_r.at[slot], sem_i).wait()

        def gather_start(sbuf, buf, sem):
            pltpu.make_async_copy(x_hbm.at[sbuf], buf, sem).start()

        def gather_wait(sbuf, buf, sem):
            pltpu.make_async_copy(x_hbm.at[sbuf], buf, sem).wait()

        def scatter(j, dbuf, buf):
            pltpu.sync_copy(buf, acc_sh.at[dbuf], add=True)
            if with_counts:
                pltpu.sync_copy(ones_v, cnt_sh.at[dbuf], add=True)

            @pl.when(j + 8 < chunks_mine)
            def _():
                idx_load(j + 8, jnp.bitwise_and(j, 7))

        @pl.when(c == 1)
        def _():
            gather_start(tail_v.at[0], rows_a, sem_a)
            gather_wait(tail_v.at[0], rows_a, sem_a)
            pltpu.sync_copy(rows_a, acc_sh.at[tail_v.at[1]], add=True)
            if with_counts:
                pltpu.sync_copy(ones_v, cnt_sh.at[tail_v.at[1]], add=True)

        n_pairs = jnp.where(c == 0, (CHUNKS_C0 - 1) // 2,
                            (CHUNKS_C1 - 1) // 2)
        idx_wait(0)
        gather_start(src_r.at[0], rows_a, sem_a)

        def step(i, carry):
            j = 2 * i
            sl_a = jnp.bitwise_and(j, 7)
            sl_b = jnp.bitwise_and(j + 1, 7)
            sl_a2 = jnp.bitwise_and(j + 2, 7)
            idx_wait(sl_b)
            gather_start(src_r.at[sl_b], rows_b, sem_b)
            gather_wait(src_r.at[sl_a], rows_a, sem_a)
            scatter(j, dst_r.at[sl_a], rows_a)
            idx_wait(sl_a2)
            gather_start(src_r.at[sl_a2], rows_a, sem_a)
            gather_wait(src_r.at[sl_b], rows_b, sem_b)
            scatter(j + 1, dst_r.at[sl_b], rows_b)
            return carry

        lax.fori_loop(0, n_pairs, step, 0)
        last_sl = jnp.bitwise_and(chunks_mine - 1, 7)
        gather_wait(src_r.at[last_sl], rows_a, sem_a)
        scatter(chunks_mine - 1, dst_r.at[last_sl], rows_a)

        plsc.subcore_barrier()
        pltpu.sync_copy(acc_sh.at[rows_sl], out_hbm.at[c, rows_sl])
        if with_counts:
            pltpu.sync_copy(cnt_sh.at[rows_sl], cnt_hbm.at[c, 0, rows_sl])

    return pl.kernel(body, mesh=mesh, out_type=out_type,
                     scratch_types=scratch)


_sc_agg_counts = _make_sc_agg(True)
_sc_agg = _make_sc_agg(False)


def _make_tc_layer(relu):
    def body(p_ref, cnt_ref, x_ref, wl_ref, b_ref, wr_ref, o_ref):
        ssum = p_ref[0] + p_ref[1]
        csum = cnt_ref[0, 0] + cnt_ref[1, 0]
        inv = 1.0 / jnp.maximum(csum, 1.0)
        mean = (ssum * lax.broadcast_in_dim(inv, (N_PAD, D), (0,)))
        mean = mean[:N_NODES]
        acc = lax.dot_general(mean, wl_ref[...], (((1,), (1,)), ((), ())),
                              preferred_element_type=jnp.float32)
        acc = acc + lax.dot_general(x_ref[...], wr_ref[...],
                                    (((1,), (1,)), ((), ())),
                                    preferred_element_type=jnp.float32)
        acc = acc + b_ref[...]
        if relu:
            acc = jnp.maximum(acc, 0.0)
        o_ref[...] = acc

    return pl.pallas_call(
        body,
        out_shape=jax.ShapeDtypeStruct((N_NODES, D), jnp.float32),
    )


_tc_layer_relu = _make_tc_layer(True)
_tc_layer_lin = _make_tc_layer(False)


def kernel(x, edge_index, W1l, b1l, W1r, W2l, b2l, W2r):
    src = edge_index[0].astype(jnp.int32)
    dst = edge_index[1].astype(jnp.int32)
    tail_src = jnp.concatenate(
        [src.reshape(16, EPW)[:, TAIL_OFF:],
         jnp.zeros((16, CHUNK - TAIL), jnp.int32)], axis=1)
    tail_dst = jnp.concatenate(
        [dst.reshape(16, EPW)[:, TAIL_OFF:],
         jnp.full((16, CHUNK - TAIL), DUMMY_ROW, jnp.int32)], axis=1)
    tails = jnp.stack([tail_src, tail_dst], axis=1)

    z2 = jnp.zeros((N_PAD, D), jnp.float32)
    z1 = jnp.zeros((N_PAD,), jnp.float32)
    b1 = b1l.reshape(1, D)
    b2 = b2l.reshape(1, D)

    p1, cnt = _sc_agg_counts(x, src, dst, tails, z2, z1)
    h = _tc_layer_relu(p1, cnt, x, W1l, b1, W1r)
    (p2,) = _sc_agg(h, src, dst, tails, z2, z1)
    return _tc_layer_lin(p2, cnt, h, W2l, b2, W2r)


# speedup vs baseline: 1.0238x; 1.0091x over previous
import jax
import jax.numpy as jnp
from jax import lax
from jax.experimental import pallas as pl
from jax.experimental.pallas import tpu as pltpu
from jax.experimental.pallas import tpu_sc as plsc

N_NODES = 10000
D = 128
E = 320000
N_PAD = 10240
EPW = E // 16
CHUNK = 128
CHUNKS_C0 = 93
CHUNKS_C1 = 63
C1_OFF = CHUNKS_C0 * CHUNK
TAIL_OFF = C1_OFF + CHUNKS_C1 * CHUNK
TAIL = EPW - TAIL_OFF
DUMMY_ROW = N_PAD - 1
ROWS_PER_SUB = N_PAD // 16


def _make_sc_agg(with_counts):
    mesh = plsc.VectorSubcoreMesh(core_axis_name="c", subcore_axis_name="s")
    out_type = [jax.ShapeDtypeStruct((2, N_PAD, D), jnp.float32)]
    scratch = [
        pltpu.VMEM((8, CHUNK), jnp.int32),
        pltpu.VMEM((8, CHUNK), jnp.int32),
        pltpu.SemaphoreType.DMA,
        pltpu.SemaphoreType.DMA,
        pltpu.VMEM((2, CHUNK), jnp.int32),
        pltpu.VMEM((CHUNK, D), jnp.float32),
        pltpu.VMEM((CHUNK, D), jnp.float32),
        pltpu.VMEM_SHARED((N_PAD, D), jnp.float32),
        pltpu.SemaphoreType.DMA,
        pltpu.SemaphoreType.DMA,
    ]
    if with_counts:
        out_type.append(jax.ShapeDtypeStruct((2, 1, N_PAD), jnp.float32))
        scratch += [
            pltpu.VMEM((CHUNK,), jnp.float32),
            pltpu.VMEM_SHARED((N_PAD,), jnp.float32),
        ]

    def body(x_hbm, src_hbm, dst_hbm, tails_hbm, z2_hbm, z1_hbm, *rest):
        if with_counts:
            (out_hbm, cnt_hbm, src_r, dst_r, sem_is, sem_id, tail_v,
             rows_a, rows_b, acc_sh, sem_a, sem_b, ones_v, cnt_sh) = rest
        else:
            (out_hbm, src_r, dst_r, sem_is, sem_id, tail_v,
             rows_a, rows_b, acc_sh, sem_a, sem_b) = rest
        c = lax.axis_index("c")
        s = lax.axis_index("s")
        row0 = s * ROWS_PER_SUB
        rows_sl = pl.ds(row0, ROWS_PER_SUB)
        base = s * EPW + c * C1_OFF
        chunks_mine = jnp.where(c == 0, CHUNKS_C0, CHUNKS_C1)
        pltpu.sync_copy(z2_hbm.at[rows_sl], acc_sh.at[rows_sl])
        if with_counts:
            pltpu.sync_copy(z1_hbm.at[rows_sl], cnt_sh.at[rows_sl])
            for i in range(CHUNK // 16):
                ones_v[pl.ds(i * 16, 16)] = jnp.ones((16,), jnp.float32)

        def idx_load(j, slot):
            sl = pl.ds(base + j * CHUNK, CHUNK)
            pltpu.make_async_copy(src_hbm.at[sl], src_r.at[slot],
                                  sem_is).start()
            pltpu.make_async_copy(dst_hbm.at[sl], dst_r.at[slot],
                                  sem_id).start()

        for t in range(8):
            idx_load(t, t)

        @pl.when(c == 1)
        def _():
            pltpu.sync_copy(tails_hbm.at[s], tail_v)
        plsc.subcore_barrier()

        def idx_wait(slot):
            pltpu.make_async_copy(src_hbm.at[pl.ds(0, CHUNK)],
                                  src_r.at[slot], sem_is).wait()
            pltpu.make_async_copy(dst_hbm.at[pl.ds(0, CHUNK)],
                                  dst_r.at[slot], sem_id).wait()

        def gather_start(sbuf, buf, sem):
            pltpu.make_async_copy(x_hbm.at[sbuf], buf, sem).start()

        def gather_wait(sbuf, buf, sem):
            pltpu.make_async_copy(x_hbm.at[sbuf], buf, sem).wait()

        def scatter(j, dbuf, buf):
            pltpu.sync_copy(buf, acc_sh.at[dbuf], add=True)
            if with_counts:
                pltpu.sync_copy(ones_v, cnt_sh.at[dbuf], add=True)

            @pl.when(j + 8 < chunks_mine)
            def _():
                idx_load(j + 8, jnp.bitwise_and(j, 7))

        @pl.when(c == 1)
        def _():
            gather_start(tail_v.at[0], rows_a, sem_a)
            gather_wait(tail_v.at[0], rows_a, sem_a)
            pltpu.sync_copy(rows_a, acc_sh.at[tail_v.at[1]], add=True)
            if with_counts:
                pltpu.sync_copy(ones_v, cnt_sh.at[tail_v.at[1]], add=True)

        n_pairs = jnp.where(c == 0, (CHUNKS_C0 - 1) // 2,
                            (CHUNKS_C1 - 1) // 2)
        idx_wait(0)
        gather_start(src_r.at[0], rows_a, sem_a)

        def step(i, carry):
            j = 2 * i
            sl_a = jnp.bitwise_and(j, 7)
            sl_b = jnp.bitwise_and(j + 1, 7)
            sl_a2 = jnp.bitwise_and(j + 2, 7)
            idx_wait(sl_b)
            gather_start(src_r.at[sl_b], rows_b, sem_b)
            gather_wait(src_r.at[sl_a], rows_a, sem_a)
            scatter(j, dst_r.at[sl_a], rows_a)
            idx_wait(sl_a2)
            gather_start(src_r.at[sl_a2], rows_a, sem_a)
            gather_wait(src_r.at[sl_b], rows_b, sem_b)
            scatter(j + 1, dst_r.at[sl_b], rows_b)
            return carry

        lax.fori_loop(0, n_pairs, step, 0)
        last_sl = jnp.bitwise_and(chunks_mine - 1, 7)
        gather_wait(src_r.at[last_sl], rows_a, sem_a)
        scatter(chunks_mine - 1, dst_r.at[last_sl], rows_a)

        plsc.subcore_barrier()
        pltpu.sync_copy(acc_sh.at[rows_sl], out_hbm.at[c, rows_sl])
        if with_counts:
            pltpu.sync_copy(cnt_sh.at[rows_sl], cnt_hbm.at[c, 0, rows_sl])

    return pl.kernel(body, mesh=mesh, out_type=out_type,
                     scratch_types=scratch)


_sc_agg_counts = _make_sc_agg(True)
_sc_agg = _make_sc_agg(False)


def _make_tc_layer(relu):
    def body(p_ref, cnt_ref, x_ref, wl_ref, b_ref, wr_ref, o_ref):
        ssum = p_ref[0] + p_ref[1]
        csum = cnt_ref[0, 0] + cnt_ref[1, 0]
        inv = 1.0 / jnp.maximum(csum, 1.0)
        mean = (ssum * lax.broadcast_in_dim(inv, (N_PAD, D), (0,)))
        mean = mean[:N_NODES]
        acc = lax.dot_general(mean, wl_ref[...], (((1,), (1,)), ((), ())),
                              preferred_element_type=jnp.float32)
        acc = acc + lax.dot_general(x_ref[...], wr_ref[...],
                                    (((1,), (1,)), ((), ())),
                                    preferred_element_type=jnp.float32)
        acc = acc + b_ref[...]
        if relu:
            acc = jnp.maximum(acc, 0.0)
        o_ref[...] = acc

    return pl.pallas_call(
        body,
        out_shape=jax.ShapeDtypeStruct((N_NODES, D), jnp.float32),
    )


_tc_layer_relu = _make_tc_layer(True)
_tc_layer_lin = _make_tc_layer(False)


def kernel(x, edge_index, W1l, b1l, W1r, W2l, b2l, W2r):
    src = edge_index[0].astype(jnp.int32)
    dst = edge_index[1].astype(jnp.int32)
    tail_src = jnp.concatenate(
        [src.reshape(16, EPW)[:, TAIL_OFF:],
         jnp.zeros((16, CHUNK - TAIL), jnp.int32)], axis=1)
    tail_dst = jnp.concatenate(
        [dst.reshape(16, EPW)[:, TAIL_OFF:],
         jnp.full((16, CHUNK - TAIL), DUMMY_ROW, jnp.int32)], axis=1)
    tails = jnp.stack([tail_src, tail_dst], axis=1)

    z2 = jnp.zeros((N_PAD, D), jnp.float32)
    z1 = jnp.zeros((N_PAD,), jnp.float32)
    b1 = b1l.reshape(1, D)
    b2 = b2l.reshape(1, D)

    p1, cnt = _sc_agg_counts(x, src, dst, tails, z2, z1)
    h = _tc_layer_relu(p1, cnt, x, W1l, b1, W1r)
    (p2,) = _sc_agg(h, src, dst, tails, z2, z1)
    return _tc_layer_lin(p2, cnt, h, W2l, b2, W2r)
